# baseline (speedup 1.0000x reference)
import functools

import jax
import jax.numpy as jnp
from jax import lax
from jax.experimental import pallas as pl
from jax.experimental.pallas import tpu as pltpu
from jax.experimental.pallas import tpu_sc as plsc

N = 10000
E = 320000
DH = 128
XW = 16
NC = 2
NS = 16
NW = NC * NS
EPW = E // NW
CH = 40
NCH = EPW // CH
SL = 640
RZ = 40
NKZ = SL // RZ

_mesh = plsc.VectorSubcoreMesh(core_axis_name="c", subcore_axis_name="s")


def _lanesum(v):
    idx = lax.iota(jnp.int32, 16)
    for sh in (8, 4, 2, 1):
        v = v + v.at[idx ^ sh].get(mode="promise_in_bounds")
    return v


NPK = N // 8
NXA = 320


RZB = 8


def _sc_edge_body(dst_h, src_h, hd_h, hs_h, xpk_h, wl_h, wc_h,
                  outM0, outM1, outX0, outX1,
                  idxd0, idxd1, idxs0, idxs1, ixa0, ixa1, idxdp0, idxdp1, idxsp0, idxsp1,
                  hdb0, hdb1, hsb0, hsb1,
                  xdb, xsb, xob, relb, wlb, wcb, shM, shX,
                  semh, semx, sems, semsx):
    idxd = (idxd0, idxd1)
    idxs = (idxs0, idxs1)
    ixa = (ixa0, ixa1)
    idxdp = (idxdp0, idxdp1)
    idxsp = (idxsp0, idxsp1)
    hdb = (hdb0, hdb1)
    hsb = (hsb0, hsb1)
    c = lax.axis_index("c")
    s = lax.axis_index("s")
    wid = s * NC + c

    pltpu.sync_copy(wl_h, wlb)
    pltpu.sync_copy(wc_h, wcb)

    zv = jnp.zeros((16,), jnp.float32)

    def _zrow(i, carry):
        for j in range(DH // 16):
            xob[i, pl.ds(16 * j, 16)] = zv
        return carry

    lax.fori_loop(0, RZB, _zrow, 0)

    row0 = s * SL

    def _zcp(k, carry):
        @pl.when(row0 + k * RZB < N)
        def _():
            pltpu.sync_copy(xob.at[pl.ds(0, RZB)],
                            shM.at[pl.ds(row0 + k * RZB, RZB)])
        return carry

    lax.fori_loop(0, SL // RZB, _zcp, 0)

    def _zcpx(k, carry):
        @pl.when(s + NS * k < NXA // RZB)
        def _():
            pltpu.sync_copy(xob.at[pl.ds(0, RZB)],
                            shX.at[pl.ds((s + NS * k) * RZB, RZB)])
        return carry

    lax.fori_loop(0, NXA // (NS * RZB) + 1, _zcpx, 0)

    plsc.subcore_barrier()

    wl = [wlb[pl.ds(16 * j, 16)] for j in range(DH // 16)]
    wc = [wcb[pl.ds(16 * j, 16)] for j in range(DH // 16)]
    lanes = lax.iota(jnp.int32, 16)
    lane3 = jnp.where(lanes == 3, 1.0, 0.0)
    not3 = jnp.where(lanes == 3, 0.0, 1.0)
    first4 = jnp.where(lanes < 4, 1.0, 0.0)
    idx3 = jnp.broadcast_to(jnp.int32(3), (16,))
    base0 = wid * EPW
    nq = CH // 16 + (1 if CH % 16 else 0)

    def _load_idx(t, b):
        base = base0 + t * CH
        pltpu.sync_copy(dst_h.at[pl.ds(base, CH)], idxdp[b].at[pl.ds(0, CH)])
        pltpu.sync_copy(src_h.at[pl.ds(base, CH)], idxsp[b].at[pl.ds(0, CH)])
        for q in range(nq):
            qb = min(16 * q, CH - 16)
            dv = idxdp[b][pl.ds(qb, 16)]
            sv = idxsp[b][pl.ds(qb, 16)]
            idxd[b][pl.ds(qb, 16)] = dv
            idxs[b][pl.ds(qb, 16)] = sv
            ixa[b][pl.ds(qb, 16)] = dv >> 5

    def _issue_h(b):
        pltpu.async_copy(hd_h.at[idxd[b]], hdb[b], semh)
        pltpu.async_copy(hs_h.at[idxs[b]], hsb[b], semh)

    def _issue_x(b):
        pltpu.async_copy(xpk_h.at[idxd[b]], xdb, semx)
        pltpu.async_copy(xpk_h.at[idxs[b]], xsb, semx)

    def _drain_h(b):
        pltpu.make_async_copy(hd_h.at[pl.ds(0, CH)], hdb[b], semh).wait()
        pltpu.make_async_copy(hd_h.at[pl.ds(0, CH)], hsb[b], semh).wait()

    def _drain_x():
        pltpu.make_async_copy(xpk_h.at[pl.ds(0, CH)], xdb, semx).wait()
        pltpu.make_async_copy(xpk_h.at[pl.ds(0, CH)], xsb, semx).wait()

    _load_idx(0, 0)
    _issue_h(0)
    _issue_x(0)

    def _chunk2(t2, carry):
        for b in (0, 1):
            t = t2 * 2 + b
            tn = t + 1

            @pl.when(tn < NCH)
            def _():
                _load_idx(tn, 1 - b)

            @pl.when(t > 0)
            def _():
                pltpu.make_async_copy(hd_h.at[pl.ds(0, CH)], xob,
                                      semsx).wait()

            _drain_x()

            @plsc.parallel_loop(0, CH, 1, unroll=4)
            def _edge_x(e):
                xd = xdb[e, pl.ds(0, 16)]
                xs = xsb[e, pl.ds(0, 16)]
                rel = xd - xs
                rr = rel * rel
                rr = rr + rr.at[lanes ^ 1].get(mode="promise_in_bounds")
                rr = rr + rr.at[lanes ^ 2].get(mode="promise_in_bounds")
                dist = rr.at[lanes & 3].get(mode="promise_in_bounds")
                relb[e, pl.ds(0, 16)] = rel + dist * lane3

            @pl.when(tn < NCH)
            def _():
                _issue_x(1 - b)

            _drain_h(b)

            @pl.when(t > 0)
            def _():
                pltpu.make_async_copy(hd_h.at[pl.ds(0, CH)], hdb[1 - b],
                                      sems).wait()

            @pl.when(tn < NCH)
            def _():
                _issue_h(1 - b)

            @plsc.parallel_loop(0, CH, 1, unroll=4)
            def _edge_m(e):
                d = idxdp[b][pl.ds(e, 16)][0]
                slot = relb[e, pl.ds(0, 16)]
                dist = slot.at[idx3].get(mode="promise_in_bounds")
                rel = slot * not3
                acc0 = jnp.zeros((16,), jnp.float32)
                acc1 = jnp.zeros((16,), jnp.float32)
                for j in range(DH // 16):
                    pre = hdb[b][e, pl.ds(16 * j, 16)] \
                        + hsb[b][e, pl.ds(16 * j, 16)] + dist * wl[j]
                    m = pre / (1.0 + jnp.exp(-pre))
                    hdb[b][e, pl.ds(16 * j, 16)] = m
                    if j % 2 == 0:
                        acc0 = acc0 + m * wc[j]
                    else:
                        acc1 = acc1 + m * wc[j]
                sd = _lanesum(acc0 + acc1)
                tv = jnp.exp(-2.0 * sd)
                cw = (1.0 - tv) / (1.0 + tv)
                val4 = (rel * cw + lane3) * first4
                o = (d & 31) * 4
                op = jnp.minimum(o, DH - 16)
                shiftv = jnp.broadcast_to(o - op, (16,))
                sv = val4.at[(lanes - shiftv) & 15].get(
                    mode="promise_in_bounds")
                for sl in range(8):
                    xob[e, pl.ds(16 * sl, 16)] = zv
                xob[e, pl.ds(op, 16)] = sv

            pltpu.async_copy(xob, shX.at[ixa[b]], semsx, add=True)
            pltpu.async_copy(hdb[b], shM.at[idxd[b]], sems, add=True)
        return carry

    lax.fori_loop(0, NCH // 2, _chunk2, 0)
    pltpu.make_async_copy(hd_h.at[pl.ds(0, CH)], xob, semsx).wait()
    pltpu.make_async_copy(hd_h.at[pl.ds(0, CH)], hdb[1], sems).wait()
    plsc.subcore_barrier()

    for k in range(NXA // (NS * RZB) + 1):
        @pl.when(s + NS * k < NXA // RZB)
        def _():
            rx = (s + NS * k) * RZB

            @pl.when(c == 0)
            def _():
                pltpu.sync_copy(shX.at[pl.ds(rx, RZB)],
                                outX0.at[pl.ds(rx, RZB)])

            @pl.when(c == 1)
            def _():
                pltpu.sync_copy(shX.at[pl.ds(rx, RZB)],
                                outX1.at[pl.ds(rx, RZB)])

    for k in range(NKZ):
        @pl.when(row0 + k * RZ < N)
        def _():
            r = row0 + k * RZ

            @pl.when(c == 0)
            def _():
                pltpu.sync_copy(shM.at[pl.ds(r, RZ)], outM0.at[pl.ds(r, RZ)])

            @pl.when(c == 1)
            def _():
                pltpu.sync_copy(shM.at[pl.ds(r, RZ)], outM1.at[pl.ds(r, RZ)])


_sc_edge = functools.partial(
    pl.kernel,
    out_type=[
        jax.ShapeDtypeStruct((N, DH), jnp.float32),
        jax.ShapeDtypeStruct((N, DH), jnp.float32),
        jax.ShapeDtypeStruct((NXA, DH), jnp.float32),
        jax.ShapeDtypeStruct((NXA, DH), jnp.float32),
    ],
    mesh=_mesh,
    scratch_types=(
        [pltpu.VMEM((CH,), jnp.int32)] * 6
        + [pltpu.VMEM((CH + 16,), jnp.int32)] * 4
        + [pltpu.VMEM((CH, DH), jnp.float32)] * 7
        + [pltpu.VMEM((CH, 32), jnp.float32)]
        + [pltpu.VMEM((DH,), jnp.float32)] * 2
        + [pltpu.VMEM_SHARED((N, DH), jnp.float32),
           pltpu.VMEM_SHARED((NXA, DH), jnp.float32),
           pltpu.SemaphoreType.DMA,
           pltpu.SemaphoreType.DMA,
           pltpu.SemaphoreType.DMA,
           pltpu.SemaphoreType.DMA]
    ),
)(_sc_edge_body)


RB = 1000
NG = N // RB

_f32 = jnp.float32


def _dot(a, b):
    return jnp.dot(a, b, preferred_element_type=_f32)


def _rspec(width):
    return pl.BlockSpec((RB, width), lambda i: (i, 0))


def _full(shape):
    return pl.BlockSpec(shape, lambda i: (0,) * len(shape))


def _k0_body(feat, we, wmt, wms, wna, bm, bn, h_o, hd_o, hs_o, a_o):
    h = _dot(feat[...], we[...])
    h_o[...] = h
    hd_o[...] = _dot(h, wmt[...]) + bm[...]
    hs_o[...] = _dot(h, wms[...])
    a_o[...] = _dot(h, wna[...]) + bn[...]


def _k0(feat, we, wmt, wms, wna, bm, bn):
    return pl.pallas_call(
        _k0_body,
        grid=(NG,),
        in_specs=[_rspec(256), _full((256, DH)), _full((DH, DH)), _full((DH, DH)),
                  _full((DH, DH)), _full((1, DH)), _full((1, DH))],
        out_specs=[_rspec(DH)] * 4,
        out_shape=[jax.ShapeDtypeStruct((N, DH), _f32)] * 4,
    )(feat, we, wmt, wms, wna, bm, bn)


def _xstep(xp, a0, a1):
    cols = lax.broadcasted_iota(jnp.int32, (RB, XW), 1)
    ax = jnp.concatenate([a0 + a1, jnp.zeros((RB, XW - 4), _f32)], axis=1)
    deg = ax[:, 3:4]
    return jnp.where(cols < 3, xp + ax / (deg + 1.0), 0.0)


def _xupd(xp, ax0, ax1):
    def body(xp_r, a0_r, a1_r, o_r):
        o_r[...] = _xstep(xp_r[...], a0_r[...], a1_r[...])

    return pl.pallas_call(
        body,
        grid=(NG,),
        in_specs=[_rspec(XW), _rspec(4), _rspec(4)],
        out_specs=_rspec(XW),
        out_shape=jax.ShapeDtypeStruct((N, XW), _f32),
    )(xp, ax0, ax1)


def _silu(x):
    return x / (1.0 + jnp.exp(-x))


def _ktr_body(xp, ax0, ax1, a, mi0, mi1, mc0, mc1, origh, wnm, wmt, wms, wnt,
              wnb, bm, bn, xp_o, hd_o, hs_o, a_o):
    xp_o[...] = _xstep(xp[...], ax0[...], ax1[...])
    av = a[...]
    c1 = _silu(av + _dot(mi0[...] + mi1[...], wnm[...]))
    c2 = _silu(av + _dot(mc0[...] + mc1[...], wnm[...]))
    hn = c2 - c1
    hd_o[...] = _dot(hn, wmt[...]) + bm[...]
    hs_o[...] = _dot(hn, wms[...])
    a_o[...] = _dot(hn, wnt[...]) + _dot(origh[...], wnb[...]) + bn[...]


def _ktr(xp, ax0, ax1, a, mi0, mi1, mc0, mc1, origh, wnm, wmt, wms, wnt, wnb,
         bm, bn):
    return pl.pallas_call(
        _ktr_body,
        grid=(NG,),
        in_specs=[_rspec(XW)] + [_rspec(4)] * 2 + [_rspec(DH)] * 6
        + [_full((DH, DH))] * 5 + [_full((1, DH))] * 2,
        out_specs=[_rspec(XW)] + [_rspec(DH)] * 3,
        out_shape=[jax.ShapeDtypeStruct((N, XW), _f32)]
        + [jax.ShapeDtypeStruct((N, DH), _f32)] * 3,
    )(xp, ax0, ax1, a, mi0, mi1, mc0, mc1, origh, wnm, wmt, wms, wnt, wnb, bm, bn)


def _kfin_body(xp, ax0, ax1, a, mi0, mi1, mc0, mc1, wnm, xp_o, h_o):
    xp_o[...] = _xstep(xp[...], ax0[...], ax1[...])
    av = a[...]
    c1 = _silu(av + _dot(mi0[...] + mi1[...], wnm[...]))
    c2 = _silu(av + _dot(mc0[...] + mc1[...], wnm[...]))
    h_o[...] = c2 - c1


def _kfin(xp, ax0, ax1, a, mi0, mi1, mc0, mc1, wnm):
    return pl.pallas_call(
        _kfin_body,
        grid=(NG,),
        in_specs=[_rspec(XW)] + [_rspec(4)] * 2 + [_rspec(DH)] * 5
        + [_full((DH, DH))],
        out_specs=[_rspec(XW), _rspec(DH)],
        out_shape=[jax.ShapeDtypeStruct((N, XW), _f32),
                   jax.ShapeDtypeStruct((N, DH), _f32)],
    )(xp, ax0, ax1, a, mi0, mi1, mc0, mc1, wnm)


B = 8
NPG = N // B


def _khead_body(h, cv, wf1, bf1, wf2, bf2, out_r):
    segs = []
    for b in range(B):
        seg = h[pl.ds(NPG * b, NPG), :]
        cvb = cv[b, :][:, None]
        segs.append(jnp.sum(seg * cvb, axis=0, keepdims=True))
    pooled = jnp.concatenate(segs, axis=0)
    denom = jnp.sum(cv[...], axis=1, keepdims=True)
    pooled = pooled / denom
    z = jnp.maximum(_dot(pooled, wf1[...]) + bf1[...], 0.0)
    o = _dot(z, wf2[...]) + bf2[...]
    out_r[...] = 1.0 / (1.0 + jnp.exp(-o))


def _khead(h, cv, wf1, bf1, wf2, bf2):
    return pl.pallas_call(
        _khead_body,
        in_specs=[pl.BlockSpec((N, DH), lambda: (0, 0)),
                  pl.BlockSpec((B, NPG), lambda: (0, 0)),
                  pl.BlockSpec((DH, DH), lambda: (0, 0)),
                  pl.BlockSpec((1, DH), lambda: (0, 0)),
                  pl.BlockSpec((DH, 1), lambda: (0, 0)),
                  pl.BlockSpec((1, 1), lambda: (0, 0))],
        out_specs=pl.BlockSpec((B, 1), lambda: (0, 0)),
        out_shape=jax.ShapeDtypeStruct((B, 1), _f32),
    )(h, cv, wf1, bf1, wf2, bf2)


def kernel(feat, coords, edge_index, cross_edge_index, c_valid, W_embed,
           Wm_0, bm_0, wc_0, Wn_0, bn_0, Wm_1, bm_1, wc_1, Wn_1, bn_1,
           Wm_2, bm_2, wc_2, Wn_2, bn_2, W_fc1, b_fc1, W_fc2, b_fc2):
    src, dst = edge_index[0], edge_index[1]
    csrc, cdst = cross_edge_index[0], cross_edge_index[1]

    Wm = [Wm_0, Wm_1, Wm_2]
    Wn = [Wn_0, Wn_1, Wn_2]
    bm = [bm_0.reshape(1, DH), bm_1.reshape(1, DH), bm_2.reshape(1, DH)]
    bn = [bn_0.reshape(1, DH), bn_1.reshape(1, DH), bn_2.reshape(1, DH)]
    wcs = [wc_0.reshape(DH), wc_1.reshape(DH), wc_2.reshape(DH)]
    wmt = [w[:DH] for w in Wm]
    wms = [w[DH:2 * DH] for w in Wm]
    wml = [w[2 * DH] for w in Wm]
    wnt = [w[:DH] for w in Wn]
    wnm = [w[DH:2 * DH] for w in Wn]
    wnb = [w[2 * DH:] for w in Wn]

    Xp = jnp.pad(coords, ((0, 0), (0, XW - 3)))

    def _pk(x):
        return jnp.pad(x, ((0, 0), (0, DH - XW)))

    def _unpk(x):
        return x.reshape(NXA * 32, 4)[:N]

    h0, Hd, Hs, A = _k0(feat, W_embed, wmt[0], wms[0], wnt[0] + wnb[0],
                        bm[0], bn[0])
    origh = h0

    for l in range(3):
        Mi0, Mi1, Xi0, Xi1 = _sc_edge(dst, src, Hd, Hs, _pk(Xp), wml[l], wcs[l])
        Xi0, Xi1 = _unpk(Xi0), _unpk(Xi1)
        Xp = _xupd(Xp, Xi0, Xi1)
        Mc0, Mc1, Xc0, Xc1 = _sc_edge(cdst, csrc, Hd, Hs, _pk(Xp), wml[l],
                                      wcs[l])
        Xc0, Xc1 = _unpk(Xc0), _unpk(Xc1)
        if l < 2:
            Xp, Hd, Hs, A = _ktr(Xp, Xc0, Xc1, A, Mi0, Mi1, Mc0, Mc1, origh,
                                 wnm[l], wmt[l + 1], wms[l + 1], wnt[l + 1],
                                 wnb[l + 1], bm[l + 1], bn[l + 1])
        else:
            Xp, h3 = _kfin(Xp, Xc0, Xc1, A, Mi0, Mi1, Mc0, Mc1, wnm[l])

    out = _khead(h3, c_valid, W_fc1, b_fc1.reshape(1, DH), W_fc2,
                 b_fc2.reshape(1, 1))
    return out.reshape(-1), Xp[:, :3]


# speedup vs baseline: 1.0155x; 1.0155x over previous
import functools

import jax
import jax.numpy as jnp
from jax import lax
from jax.experimental import pallas as pl
from jax.experimental.pallas import tpu as pltpu
from jax.experimental.pallas import tpu_sc as plsc

N = 10000
E = 320000
DH = 128
XW = 16
NC = 2
NS = 16
NW = NC * NS
EPW = E // NW
CH = 40
NCH = EPW // CH
SL = 640
RZ = 40
NKZ = SL // RZ

_mesh = plsc.VectorSubcoreMesh(core_axis_name="c", subcore_axis_name="s")


def _lanesum(v):
    idx = lax.iota(jnp.int32, 16)
    for sh in (8, 4, 2, 1):
        v = v + v.at[idx ^ sh].get(mode="promise_in_bounds")
    return v


NPK = N // 8
NXA = 320


RZB = 40


def _sc_edge_body(dst_h, src_h, hd_h, hs_h, xpk_h, wl_h, wc_h,
                  outM0, outM1, outX0, outX1,
                  idxd0, idxd1, idxs0, idxs1, ixa0, ixa1, idxdp0, idxdp1, idxsp0, idxsp1,
                  hdb0, hdb1, hsb0, hsb1,
                  xdb, xsb, xob, relb, wlb, wcb, shM, shX,
                  semh, semx, sems, semsx):
    idxd = (idxd0, idxd1)
    idxs = (idxs0, idxs1)
    ixa = (ixa0, ixa1)
    idxdp = (idxdp0, idxdp1)
    idxsp = (idxsp0, idxsp1)
    hdb = (hdb0, hdb1)
    hsb = (hsb0, hsb1)
    c = lax.axis_index("c")
    s = lax.axis_index("s")
    wid = s * NC + c

    pltpu.sync_copy(wl_h, wlb)
    pltpu.sync_copy(wc_h, wcb)

    zv = jnp.zeros((16,), jnp.float32)

    def _zrow(i, carry):
        for j in range(DH // 16):
            xob[i, pl.ds(16 * j, 16)] = zv
        return carry

    lax.fori_loop(0, RZB, _zrow, 0)

    row0 = s * SL

    def _zcp(k, carry):
        @pl.when(row0 + k * RZB < N)
        def _():
            pltpu.sync_copy(xob.at[pl.ds(0, RZB)],
                            shM.at[pl.ds(row0 + k * RZB, RZB)])
        return carry

    lax.fori_loop(0, SL // RZB, _zcp, 0)

    def _zcpx(k, carry):
        @pl.when(s + NS * k < NXA // RZB)
        def _():
            pltpu.sync_copy(xob.at[pl.ds(0, RZB)],
                            shX.at[pl.ds((s + NS * k) * RZB, RZB)])
        return carry

    lax.fori_loop(0, NXA // (NS * RZB) + 1, _zcpx, 0)

    plsc.subcore_barrier()

    wl = [wlb[pl.ds(16 * j, 16)] for j in range(DH // 16)]
    wc = [wcb[pl.ds(16 * j, 16)] for j in range(DH // 16)]
    lanes = lax.iota(jnp.int32, 16)
    lane3 = jnp.where(lanes == 3, 1.0, 0.0)
    not3 = jnp.where(lanes == 3, 0.0, 1.0)
    first4 = jnp.where(lanes < 4, 1.0, 0.0)
    idx3 = jnp.broadcast_to(jnp.int32(3), (16,))
    base0 = wid * EPW
    nq = CH // 16 + (1 if CH % 16 else 0)

    def _load_idx(t, b):
        base = base0 + t * CH
        pltpu.sync_copy(dst_h.at[pl.ds(base, CH)], idxdp[b].at[pl.ds(0, CH)])
        pltpu.sync_copy(src_h.at[pl.ds(base, CH)], idxsp[b].at[pl.ds(0, CH)])
        for q in range(nq):
            qb = min(16 * q, CH - 16)
            dv = idxdp[b][pl.ds(qb, 16)]
            sv = idxsp[b][pl.ds(qb, 16)]
            idxd[b][pl.ds(qb, 16)] = dv
            idxs[b][pl.ds(qb, 16)] = sv
            ixa[b][pl.ds(qb, 16)] = dv >> 5

    def _issue_h(b):
        pltpu.async_copy(hd_h.at[idxd[b]], hdb[b], semh)
        pltpu.async_copy(hs_h.at[idxs[b]], hsb[b], semh)

    def _issue_x(b):
        pltpu.async_copy(xpk_h.at[idxd[b]], xdb, semx)
        pltpu.async_copy(xpk_h.at[idxs[b]], xsb, semx)

    def _drain_h(b):
        pltpu.make_async_copy(hd_h.at[pl.ds(0, CH)], hdb[b], semh).wait()
        pltpu.make_async_copy(hd_h.at[pl.ds(0, CH)], hsb[b], semh).wait()

    def _drain_x():
        pltpu.make_async_copy(xpk_h.at[pl.ds(0, CH)], xdb, semx).wait()
        pltpu.make_async_copy(xpk_h.at[pl.ds(0, CH)], xsb, semx).wait()

    _load_idx(0, 0)
    _issue_h(0)
    _issue_x(0)

    def _chunk2(t2, carry):
        for b in (0, 1):
            t = t2 * 2 + b
            tn = t + 1

            @pl.when(tn < NCH)
            def _():
                _load_idx(tn, 1 - b)

            @pl.when(t > 0)
            def _():
                pltpu.make_async_copy(hd_h.at[pl.ds(0, CH)], xob,
                                      semsx).wait()

            _drain_x()

            @plsc.parallel_loop(0, CH, 1, unroll=4)
            def _edge_x(e):
                xd = xdb[e, pl.ds(0, 16)]
                xs = xsb[e, pl.ds(0, 16)]
                rel = xd - xs
                rr = rel * rel
                rr = rr + rr.at[lanes ^ 1].get(mode="promise_in_bounds")
                rr = rr + rr.at[lanes ^ 2].get(mode="promise_in_bounds")
                dist = rr.at[lanes & 3].get(mode="promise_in_bounds")
                relb[e, pl.ds(0, 16)] = rel + dist * lane3

            @pl.when(tn < NCH)
            def _():
                _issue_x(1 - b)

            _drain_h(b)

            @pl.when(t > 0)
            def _():
                pltpu.make_async_copy(hd_h.at[pl.ds(0, CH)], hdb[1 - b],
                                      sems).wait()

            @pl.when(tn < NCH)
            def _():
                _issue_h(1 - b)

            @plsc.parallel_loop(0, CH, 1, unroll=2)
            def _edge_m(e):
                d = idxdp[b][pl.ds(e, 16)][0]
                slot = relb[e, pl.ds(0, 16)]
                dist = slot.at[idx3].get(mode="promise_in_bounds")
                rel = slot * not3
                acc0 = jnp.zeros((16,), jnp.float32)
                acc1 = jnp.zeros((16,), jnp.float32)
                for j in range(DH // 16):
                    pre = hdb[b][e, pl.ds(16 * j, 16)] \
                        + hsb[b][e, pl.ds(16 * j, 16)] + dist * wl[j]
                    m = pre / (1.0 + jnp.exp(-pre))
                    hdb[b][e, pl.ds(16 * j, 16)] = m
                    if j % 2 == 0:
                        acc0 = acc0 + m * wc[j]
                    else:
                        acc1 = acc1 + m * wc[j]
                sd = _lanesum(acc0 + acc1)
                tv = jnp.exp(-2.0 * sd)
                cw = (1.0 - tv) / (1.0 + tv)
                val4 = (rel * cw + lane3) * first4
                o = (d & 31) * 4
                op = jnp.minimum(o, DH - 16)
                shiftv = jnp.broadcast_to(o - op, (16,))
                sv = val4.at[(lanes - shiftv) & 15].get(
                    mode="promise_in_bounds")
                for sl in range(8):
                    xob[e, pl.ds(16 * sl, 16)] = zv
                xob[e, pl.ds(op, 16)] = sv

            pltpu.async_copy(xob, shX.at[ixa[b]], semsx, add=True)
            pltpu.async_copy(hdb[b], shM.at[idxd[b]], sems, add=True)
        return carry

    lax.fori_loop(0, NCH // 2, _chunk2, 0)
    pltpu.make_async_copy(hd_h.at[pl.ds(0, CH)], xob, semsx).wait()
    pltpu.make_async_copy(hd_h.at[pl.ds(0, CH)], hdb[1], sems).wait()
    plsc.subcore_barrier()

    for k in range(NXA // (NS * RZB) + 1):
        @pl.when(s + NS * k < NXA // RZB)
        def _():
            rx = (s + NS * k) * RZB

            @pl.when(c == 0)
            def _():
                pltpu.sync_copy(shX.at[pl.ds(rx, RZB)],
                                outX0.at[pl.ds(rx, RZB)])

            @pl.when(c == 1)
            def _():
                pltpu.sync_copy(shX.at[pl.ds(rx, RZB)],
                                outX1.at[pl.ds(rx, RZB)])

    for k in range(NKZ):
        @pl.when(row0 + k * RZ < N)
        def _():
            r = row0 + k * RZ

            @pl.when(c == 0)
            def _():
                pltpu.sync_copy(shM.at[pl.ds(r, RZ)], outM0.at[pl.ds(r, RZ)])

            @pl.when(c == 1)
            def _():
                pltpu.sync_copy(shM.at[pl.ds(r, RZ)], outM1.at[pl.ds(r, RZ)])


_sc_edge = functools.partial(
    pl.kernel,
    out_type=[
        jax.ShapeDtypeStruct((N, DH), jnp.float32),
        jax.ShapeDtypeStruct((N, DH), jnp.float32),
        jax.ShapeDtypeStruct((NXA, DH), jnp.float32),
        jax.ShapeDtypeStruct((NXA, DH), jnp.float32),
    ],
    mesh=_mesh,
    scratch_types=(
        [pltpu.VMEM((CH,), jnp.int32)] * 6
        + [pltpu.VMEM((CH + 16,), jnp.int32)] * 4
        + [pltpu.VMEM((CH, DH), jnp.float32)] * 7
        + [pltpu.VMEM((CH, 32), jnp.float32)]
        + [pltpu.VMEM((DH,), jnp.float32)] * 2
        + [pltpu.VMEM_SHARED((N, DH), jnp.float32),
           pltpu.VMEM_SHARED((NXA, DH), jnp.float32),
           pltpu.SemaphoreType.DMA,
           pltpu.SemaphoreType.DMA,
           pltpu.SemaphoreType.DMA,
           pltpu.SemaphoreType.DMA]
    ),
)(_sc_edge_body)


RB = 1000
NG = N // RB

_f32 = jnp.float32


def _dot(a, b):
    return jnp.dot(a, b, preferred_element_type=_f32)


def _rspec(width):
    return pl.BlockSpec((RB, width), lambda i: (i, 0))


def _full(shape):
    return pl.BlockSpec(shape, lambda i: (0,) * len(shape))


def _k0_body(feat, we, wmt, wms, wna, bm, bn, h_o, hd_o, hs_o, a_o):
    h = _dot(feat[...], we[...])
    h_o[...] = h
    hd_o[...] = _dot(h, wmt[...]) + bm[...]
    hs_o[...] = _dot(h, wms[...])
    a_o[...] = _dot(h, wna[...]) + bn[...]


def _k0(feat, we, wmt, wms, wna, bm, bn):
    return pl.pallas_call(
        _k0_body,
        grid=(NG,),
        in_specs=[_rspec(256), _full((256, DH)), _full((DH, DH)), _full((DH, DH)),
                  _full((DH, DH)), _full((1, DH)), _full((1, DH))],
        out_specs=[_rspec(DH)] * 4,
        out_shape=[jax.ShapeDtypeStruct((N, DH), _f32)] * 4,
    )(feat, we, wmt, wms, wna, bm, bn)


def _xstep(xp, a0, a1):
    cols = lax.broadcasted_iota(jnp.int32, (RB, XW), 1)
    ax = jnp.concatenate([a0 + a1, jnp.zeros((RB, XW - 4), _f32)], axis=1)
    deg = ax[:, 3:4]
    return jnp.where(cols < 3, xp + ax / (deg + 1.0), 0.0)


def _xupd(xp, ax0, ax1):
    def body(xp_r, a0_r, a1_r, o_r):
        o_r[...] = _xstep(xp_r[...], a0_r[...], a1_r[...])

    return pl.pallas_call(
        body,
        grid=(NG,),
        in_specs=[_rspec(XW), _rspec(4), _rspec(4)],
        out_specs=_rspec(XW),
        out_shape=jax.ShapeDtypeStruct((N, XW), _f32),
    )(xp, ax0, ax1)


def _silu(x):
    return x / (1.0 + jnp.exp(-x))


def _ktr_body(xp, ax0, ax1, a, mi0, mi1, mc0, mc1, origh, wnm, wmt, wms, wnt,
              wnb, bm, bn, xp_o, hd_o, hs_o, a_o):
    xp_o[...] = _xstep(xp[...], ax0[...], ax1[...])
    av = a[...]
    c1 = _silu(av + _dot(mi0[...] + mi1[...], wnm[...]))
    c2 = _silu(av + _dot(mc0[...] + mc1[...], wnm[...]))
    hn = c2 - c1
    hd_o[...] = _dot(hn, wmt[...]) + bm[...]
    hs_o[...] = _dot(hn, wms[...])
    a_o[...] = _dot(hn, wnt[...]) + _dot(origh[...], wnb[...]) + bn[...]


def _ktr(xp, ax0, ax1, a, mi0, mi1, mc0, mc1, origh, wnm, wmt, wms, wnt, wnb,
         bm, bn):
    return pl.pallas_call(
        _ktr_body,
        grid=(NG,),
        in_specs=[_rspec(XW)] + [_rspec(4)] * 2 + [_rspec(DH)] * 6
        + [_full((DH, DH))] * 5 + [_full((1, DH))] * 2,
        out_specs=[_rspec(XW)] + [_rspec(DH)] * 3,
        out_shape=[jax.ShapeDtypeStruct((N, XW), _f32)]
        + [jax.ShapeDtypeStruct((N, DH), _f32)] * 3,
    )(xp, ax0, ax1, a, mi0, mi1, mc0, mc1, origh, wnm, wmt, wms, wnt, wnb, bm, bn)


def _kfin_body(xp, ax0, ax1, a, mi0, mi1, mc0, mc1, wnm, xp_o, h_o):
    xp_o[...] = _xstep(xp[...], ax0[...], ax1[...])
    av = a[...]
    c1 = _silu(av + _dot(mi0[...] + mi1[...], wnm[...]))
    c2 = _silu(av + _dot(mc0[...] + mc1[...], wnm[...]))
    h_o[...] = c2 - c1


def _kfin(xp, ax0, ax1, a, mi0, mi1, mc0, mc1, wnm):
    return pl.pallas_call(
        _kfin_body,
        grid=(NG,),
        in_specs=[_rspec(XW)] + [_rspec(4)] * 2 + [_rspec(DH)] * 5
        + [_full((DH, DH))],
        out_specs=[_rspec(XW), _rspec(DH)],
        out_shape=[jax.ShapeDtypeStruct((N, XW), _f32),
                   jax.ShapeDtypeStruct((N, DH), _f32)],
    )(xp, ax0, ax1, a, mi0, mi1, mc0, mc1, wnm)


B = 8
NPG = N // B


def _khead_body(h, cv, wf1, bf1, wf2, bf2, out_r):
    segs = []
    for b in range(B):
        seg = h[pl.ds(NPG * b, NPG), :]
        cvb = cv[b, :][:, None]
        segs.append(jnp.sum(seg * cvb, axis=0, keepdims=True))
    pooled = jnp.concatenate(segs, axis=0)
    denom = jnp.sum(cv[...], axis=1, keepdims=True)
    pooled = pooled / denom
    z = jnp.maximum(_dot(pooled, wf1[...]) + bf1[...], 0.0)
    o = _dot(z, wf2[...]) + bf2[...]
    out_r[...] = 1.0 / (1.0 + jnp.exp(-o))


def _khead(h, cv, wf1, bf1, wf2, bf2):
    return pl.pallas_call(
        _khead_body,
        in_specs=[pl.BlockSpec((N, DH), lambda: (0, 0)),
                  pl.BlockSpec((B, NPG), lambda: (0, 0)),
                  pl.BlockSpec((DH, DH), lambda: (0, 0)),
                  pl.BlockSpec((1, DH), lambda: (0, 0)),
                  pl.BlockSpec((DH, 1), lambda: (0, 0)),
                  pl.BlockSpec((1, 1), lambda: (0, 0))],
        out_specs=pl.BlockSpec((B, 1), lambda: (0, 0)),
        out_shape=jax.ShapeDtypeStruct((B, 1), _f32),
    )(h, cv, wf1, bf1, wf2, bf2)


def kernel(feat, coords, edge_index, cross_edge_index, c_valid, W_embed,
           Wm_0, bm_0, wc_0, Wn_0, bn_0, Wm_1, bm_1, wc_1, Wn_1, bn_1,
           Wm_2, bm_2, wc_2, Wn_2, bn_2, W_fc1, b_fc1, W_fc2, b_fc2):
    src, dst = edge_index[0], edge_index[1]
    csrc, cdst = cross_edge_index[0], cross_edge_index[1]

    Wm = [Wm_0, Wm_1, Wm_2]
    Wn = [Wn_0, Wn_1, Wn_2]
    bm = [bm_0.reshape(1, DH), bm_1.reshape(1, DH), bm_2.reshape(1, DH)]
    bn = [bn_0.reshape(1, DH), bn_1.reshape(1, DH), bn_2.reshape(1, DH)]
    wcs = [wc_0.reshape(DH), wc_1.reshape(DH), wc_2.reshape(DH)]
    wmt = [w[:DH] for w in Wm]
    wms = [w[DH:2 * DH] for w in Wm]
    wml = [w[2 * DH] for w in Wm]
    wnt = [w[:DH] for w in Wn]
    wnm = [w[DH:2 * DH] for w in Wn]
    wnb = [w[2 * DH:] for w in Wn]

    Xp = jnp.pad(coords, ((0, 0), (0, XW - 3)))

    def _pk(x):
        return jnp.pad(x, ((0, 0), (0, DH - XW)))

    def _unpk(x):
        return x.reshape(NXA * 32, 4)[:N]

    h0, Hd, Hs, A = _k0(feat, W_embed, wmt[0], wms[0], wnt[0] + wnb[0],
                        bm[0], bn[0])
    origh = h0

    for l in range(3):
        Mi0, Mi1, Xi0, Xi1 = _sc_edge(dst, src, Hd, Hs, _pk(Xp), wml[l], wcs[l])
        Xi0, Xi1 = _unpk(Xi0), _unpk(Xi1)
        Xp = _xupd(Xp, Xi0, Xi1)
        Mc0, Mc1, Xc0, Xc1 = _sc_edge(cdst, csrc, Hd, Hs, _pk(Xp), wml[l],
                                      wcs[l])
        Xc0, Xc1 = _unpk(Xc0), _unpk(Xc1)
        if l < 2:
            Xp, Hd, Hs, A = _ktr(Xp, Xc0, Xc1, A, Mi0, Mi1, Mc0, Mc1, origh,
                                 wnm[l], wmt[l + 1], wms[l + 1], wnt[l + 1],
                                 wnb[l + 1], bm[l + 1], bn[l + 1])
        else:
            Xp, h3 = _kfin(Xp, Xc0, Xc1, A, Mi0, Mi1, Mc0, Mc1, wnm[l])

    out = _khead(h3, c_valid, W_fc1, b_fc1.reshape(1, DH), W_fc2,
                 b_fc2.reshape(1, 1))
    return out.reshape(-1), Xp[:, :3]
